# Initial kernel scaffold; baseline (speedup 1.0000x reference)
import jax
import jax.numpy as jnp
from jax.experimental import pallas as pl


def kernel(x, W):
    raise NotImplementedError("write your pallas kernel here")


# speedup vs baseline: 1.3521x; 1.3521x over previous
import functools
import math

import jax
import jax.numpy as jnp
from jax import lax
from jax.experimental import pallas as pl
from jax.experimental.pallas import tpu as pltpu, tpu_sc as plsc

VOCAB_ROWS = 54
EMB_DIM = 512
SCALE = math.sqrt(float(EMB_DIM))

NC = 2
NS = 16
NW = NC * NS

B_TOTAL = 1024 * 200
B_PER_W = B_TOTAL // NW
CHUNK = 32
NBUF = 4
CHUNKS_PER_W = B_PER_W // CHUNK
ITERS = CHUNKS_PER_W // NBUF

W_FLAT = VOCAB_ROWS * EMB_DIM
W_PER_W = W_FLAT // NW

_mesh = plsc.VectorSubcoreMesh(core_axis_name="c", subcore_axis_name="s")


@functools.partial(
    pl.kernel,
    mesh=_mesh,
    out_type=jax.ShapeDtypeStruct((W_FLAT,), jnp.float32),
    scratch_types=[pltpu.VMEM((W_PER_W,), jnp.float32)],
)
def _scale_table(w_hbm, out_hbm, w_v):
    wid = lax.axis_index("s") * NC + lax.axis_index("c")
    base = wid * W_PER_W
    pltpu.sync_copy(w_hbm.at[pl.ds(base, W_PER_W)], w_v)
    for j in range(W_PER_W // 16):
        w_v[pl.ds(j * 16, 16)] = w_v[pl.ds(j * 16, 16)] * SCALE
    pltpu.sync_copy(w_v, out_hbm.at[pl.ds(base, W_PER_W)])


@functools.partial(
    pl.kernel,
    mesh=_mesh,
    out_type=jax.ShapeDtypeStruct((B_TOTAL, EMB_DIM), jnp.float32),
    scratch_types=(
        [pltpu.VMEM((B_PER_W,), jnp.int32)]
        + [pltpu.VMEM((CHUNK, EMB_DIM), jnp.float32) for _ in range(NBUF)]
        + [pltpu.SemaphoreType.DMA for _ in range(2 * NBUF)]
    ),
)
def _gather(w_hbm, x_hbm, out_hbm, idx_v, *rest):
    bufs = list(rest[:NBUF])
    gsems = list(rest[NBUF:2 * NBUF])
    ssems = list(rest[2 * NBUF:])

    wid = lax.axis_index("s") * NC + lax.axis_index("c")
    base = wid * B_PER_W
    pltpu.sync_copy(x_hbm.at[pl.ds(base, B_PER_W)], idx_v)

    def body(i, carry):
        c0 = i * NBUF
        gathers = []
        for b in range(NBUF):
            @pl.when(i > 0)
            def _wait_store(b=b):
                pltpu.make_async_copy(
                    bufs[b], out_hbm.at[pl.ds(base, CHUNK)], ssems[b]
                ).wait()
            idx_slice = idx_v.at[pl.ds((c0 + b) * CHUNK, CHUNK)]
            g = pltpu.make_async_copy(w_hbm.at[idx_slice], bufs[b], gsems[b])
            g.start()
            gathers.append(g)
        for b in range(NBUF):
            gathers[b].wait()
            pltpu.make_async_copy(
                bufs[b],
                out_hbm.at[pl.ds(base + (c0 + b) * CHUNK, CHUNK)],
                ssems[b],
            ).start()
        return carry

    lax.fori_loop(0, ITERS, body, 0)
    for b in range(NBUF):
        pltpu.make_async_copy(
            bufs[b], out_hbm.at[pl.ds(base, CHUNK)], ssems[b]
        ).wait()


def kernel(x, W):
    w_scaled = _scale_table(W.reshape(-1)).reshape(VOCAB_ROWS, EMB_DIM)
    out = _gather(w_scaled, x.reshape(-1).astype(jnp.int32))
    return out.reshape(x.shape[0], x.shape[1], EMB_DIM)
